# Initial kernel scaffold; baseline (speedup 1.0000x reference)
import jax
import jax.numpy as jnp
from jax.experimental import pallas as pl


def kernel(x, edge_index, edge_attr, pos, nw0, nb0, nw1, nb1, ew0, eb0, ew1, eb1, gw0, gb0, gw1, gb1):
    raise NotImplementedError("write your pallas kernel here")


# speedup vs baseline: 6.5376x; 6.5376x over previous
import functools

import jax
import jax.numpy as jnp
from jax import lax
from jax.experimental import pallas as pl
from jax.experimental.pallas import tpu as pltpu
from jax.experimental.pallas import tpu_sc as plsc

N = 10000
E = 320000
H = 128
NPAD = 10240
CH = 128
NW = 32
EPAD = 327680
NCH = EPAD // (NW * CH)
RPS = NPAD // 16


def _elu(v):
    return jnp.where(v > 0, v, jnp.exp(jnp.minimum(v, 0.0)) - 1.0)


def _edge_mlp_body(a_ref, ew0_ref, eb0_ref, ew1r_ref, eb1_ref, o_ref):
    a = a_ref[...]
    h1 = _elu(a * ew0_ref[...] + eb0_ref[...])
    w = jnp.sum(h1 * ew1r_ref[...], axis=1, keepdims=True) + eb1_ref[...]
    o_ref[...] = _elu(w)


def _edge_mlp(edge_attr, ew0, eb0, ew1, eb1):
    BE = 2000
    grid = (E // BE,)
    return pl.pallas_call(
        _edge_mlp_body,
        grid=grid,
        in_specs=[
            pl.BlockSpec((BE, 1), lambda i: (i, 0)),
            pl.BlockSpec((1, H), lambda i: (0, 0)),
            pl.BlockSpec((1, H), lambda i: (0, 0)),
            pl.BlockSpec((1, H), lambda i: (0, 0)),
            pl.BlockSpec((1, 1), lambda i: (0, 0)),
        ],
        out_specs=pl.BlockSpec((BE, 1), lambda i: (i, 0)),
        out_shape=jax.ShapeDtypeStruct((E, 1), jnp.float32),
    )(edge_attr, ew0, eb0.reshape(1, H), ew1.reshape(1, H), eb1.reshape(1, 1))


def _prep_body(x_ref, nw0_ref, nb0_ref, nw1_ref, nb1_ref, gw0_ref,
               dp0_ref, dp1_ref, g_ref, self_ref, dis_ref, invd_ref):
    h = _elu(jnp.dot(x_ref[...], nw0_ref[...],
                     preferred_element_type=jnp.float32) + nb0_ref[...])
    h = _elu(jnp.dot(h, nw1_ref[...],
                     preferred_element_type=jnp.float32) + nb1_ref[...])
    hw = jnp.dot(h, gw0_ref[...], preferred_element_type=jnp.float32)
    deg = 1.0 + dp0_ref[...] + dp1_ref[...]
    pos = deg > 0
    dis = jnp.where(pos, lax.rsqrt(jnp.abs(deg)), 0.0)
    invd = jnp.where(pos, 1.0 / deg, 0.0)
    g_ref[...] = hw * dis
    self_ref[...] = hw * invd
    dis_ref[...] = dis
    invd_ref[...] = invd


def _prep(x, nw0, nb0, nw1, nb1, gw0, dp0, dp1):
    BR = 400
    grid = (N // BR,)
    r = pl.BlockSpec((BR, H), lambda i: (i, 0))
    wsp = pl.BlockSpec((H, H), lambda i: (0, 0))
    b = pl.BlockSpec((1, H), lambda i: (0, 0))
    c = pl.BlockSpec((BR, 1), lambda i: (i, 0))
    return pl.pallas_call(
        _prep_body,
        grid=grid,
        in_specs=[r, wsp, b, wsp, b, wsp, c, c],
        out_specs=[r, r, c, c],
        out_shape=[
            jax.ShapeDtypeStruct((N, H), jnp.float32),
            jax.ShapeDtypeStruct((N, H), jnp.float32),
            jax.ShapeDtypeStruct((N, 1), jnp.float32),
            jax.ShapeDtypeStruct((N, 1), jnp.float32),
        ],
    )(x, nw0, nb0.reshape(1, H), nw1, nb1.reshape(1, H), gw0, dp0, dp1)


def _mid_body(q0_ref, q1_ref, self_ref, dis_ref, invd_ref, gw_ref, gb_ref,
              g_ref, selfo_ref):
    msum = q0_ref[0] + q1_ref[0]
    out1 = _elu(dis_ref[...] * msum + self_ref[...] + gb_ref[...])
    hw = jnp.dot(out1, gw_ref[...], preferred_element_type=jnp.float32)
    g_ref[...] = hw * dis_ref[...]
    selfo_ref[...] = hw * invd_ref[...]


def _mid(q, self0, dis, invd, gw, gb):
    BR = 400
    grid = (N // BR,)
    q0 = pl.BlockSpec((1, BR, H), lambda i: (0, i, 0))
    q1 = pl.BlockSpec((1, BR, H), lambda i: (1, i, 0))
    r = pl.BlockSpec((BR, H), lambda i: (i, 0))
    wsp = pl.BlockSpec((H, H), lambda i: (0, 0))
    b = pl.BlockSpec((1, H), lambda i: (0, 0))
    c = pl.BlockSpec((BR, 1), lambda i: (i, 0))
    return pl.pallas_call(
        _mid_body,
        grid=grid,
        in_specs=[q0, q1, r, c, c, wsp, b],
        out_specs=[r, r],
        out_shape=[
            jax.ShapeDtypeStruct((N, H), jnp.float32),
            jax.ShapeDtypeStruct((N, H), jnp.float32),
        ],
    )(q, q, self0, dis, invd, gw, gb.reshape(1, H))


def _final_body(q0_ref, q1_ref, self_ref, dis_ref, gb_ref, o_ref):
    msum = q0_ref[0] + q1_ref[0]
    o_ref[...] = _elu(dis_ref[...] * msum + self_ref[...] + gb_ref[...])


def _final(q, self1, dis, gb):
    BR = 400
    grid = (N // BR,)
    q0 = pl.BlockSpec((1, BR, H), lambda i: (0, i, 0))
    q1 = pl.BlockSpec((1, BR, H), lambda i: (1, i, 0))
    r = pl.BlockSpec((BR, H), lambda i: (i, 0))
    b = pl.BlockSpec((1, H), lambda i: (0, 0))
    c = pl.BlockSpec((BR, 1), lambda i: (i, 0))
    return pl.pallas_call(
        _final_body,
        grid=grid,
        in_specs=[q0, q1, r, c, b],
        out_specs=r,
        out_shape=jax.ShapeDtypeStruct((N, H), jnp.float32),
    )(q, q, self1, dis, gb.reshape(1, H))


_MESH = plsc.VectorSubcoreMesh(core_axis_name="c", subcore_axis_name="s")


@functools.partial(
    pl.kernel,
    mesh=_MESH,
    out_type=jax.ShapeDtypeStruct((2, NPAD), jnp.float32),
    scratch_types=[
        pltpu.VMEM((NCH, CH), jnp.int32),
        pltpu.VMEM((NCH, CH), jnp.float32),
        pltpu.VMEM((CH,), jnp.float32),
        pltpu.VMEM_SHARED((NPAD,), jnp.float32),
    ],
)
def _deg_kernel(dst_hbm, w_hbm, out_hbm, didx_v, w_v, zbuf, acc_sh):
    cid = lax.axis_index("c")
    sid = lax.axis_index("s")
    ws = cid * 16 + sid
    pltpu.sync_copy(dst_hbm.at[pl.ds(ws * NCH, NCH)], didx_v)
    pltpu.sync_copy(w_hbm.at[pl.ds(ws * NCH, NCH)], w_v)
    for j in range(CH // 16):
        zbuf[pl.ds(j * 16, 16)] = jnp.zeros((16,), jnp.float32)
    for k in range(RPS // CH):
        pltpu.sync_copy(zbuf, acc_sh.at[pl.ds(sid * RPS + k * CH, CH)])
    plsc.subcore_barrier()

    def body(c, carry):
        pltpu.sync_copy(w_v.at[c], acc_sh.at[didx_v.at[c]], add=True)
        return carry

    lax.fori_loop(0, NCH, body, 0)
    plsc.subcore_barrier()
    pltpu.sync_copy(acc_sh.at[pl.ds(sid * RPS, RPS)],
                    out_hbm.at[cid, pl.ds(sid * RPS, RPS)])


@functools.partial(
    pl.kernel,
    mesh=_MESH,
    out_type=jax.ShapeDtypeStruct((2, NPAD, H), jnp.float32),
    scratch_types=[
        pltpu.VMEM((NCH, CH), jnp.int32),
        pltpu.VMEM((NCH, CH), jnp.int32),
        pltpu.VMEM((NCH, CH), jnp.float32),
        pltpu.VMEM((CH, H), jnp.float32),
        pltpu.VMEM_SHARED((NPAD, H), jnp.float32),
        pltpu.SemaphoreType.DMA,
    ],
)
def _msg_kernel(g_hbm, src_hbm, dst_hbm, w_hbm, out_hbm,
                sidx_v, didx_v, w_v, rows_v, acc_sh, sem):
    cid = lax.axis_index("c")
    sid = lax.axis_index("s")
    ws = cid * 16 + sid
    pltpu.sync_copy(src_hbm.at[pl.ds(ws * NCH, NCH)], sidx_v)
    pltpu.sync_copy(dst_hbm.at[pl.ds(ws * NCH, NCH)], didx_v)
    pltpu.sync_copy(w_hbm.at[pl.ds(ws * NCH, NCH)], w_v)

    def zrow(r, carry):
        for j in range(H // 16):
            rows_v[r, pl.ds(j * 16, 16)] = jnp.zeros((16,), jnp.float32)
        return carry

    lax.fori_loop(0, CH, zrow, 0)
    for k in range(RPS // CH):
        pltpu.sync_copy(rows_v, acc_sh.at[pl.ds(sid * RPS + k * CH, CH)])
    plsc.subcore_barrier()

    def body(c, carry):
        pltpu.async_copy(g_hbm.at[sidx_v.at[c]], rows_v, sem).wait()

        def rbody(eg, rc):
            wvec = w_v[c, pl.ds(eg * 16, 16)]
            for i in range(16):
                s = wvec[i]
                e = eg * 16 + i
                for j in range(H // 16):
                    rows_v[e, pl.ds(j * 16, 16)] = (
                        rows_v[e, pl.ds(j * 16, 16)] * s)
            return rc

        lax.fori_loop(0, CH // 16, rbody, 0)
        pltpu.sync_copy(rows_v, acc_sh.at[didx_v.at[c]], add=True)
        return carry

    lax.fori_loop(0, NCH, body, 0)
    plsc.subcore_barrier()
    pltpu.sync_copy(acc_sh.at[pl.ds(sid * RPS, RPS)],
                    out_hbm.at[cid, pl.ds(sid * RPS, RPS)])


def kernel(x, edge_index, edge_attr, pos, nw0, nb0, nw1, nb1,
           ew0, eb0, ew1, eb1, gw0, gb0, gw1, gb1):
    pad = jnp.zeros((EPAD - E,), jnp.int32)
    src2 = jnp.concatenate([edge_index[0], pad]).reshape(EPAD // CH, CH)
    dst2 = jnp.concatenate([edge_index[1], pad]).reshape(EPAD // CH, CH)

    w_edge = _edge_mlp(edge_attr, ew0, eb0, ew1, eb1)
    w2 = jnp.concatenate(
        [w_edge.reshape(E), jnp.zeros((EPAD - E,), jnp.float32)]
    ).reshape(EPAD // CH, CH)

    degp = _deg_kernel(dst2, w2)
    dp0 = degp[0, :N].reshape(N, 1)
    dp1 = degp[1, :N].reshape(N, 1)

    g0, self0, dis, invd = _prep(x, nw0, nb0, nw1, nb1, gw0, dp0, dp1)

    q = _msg_kernel(g0, src2, dst2, w2)
    g1, self1 = _mid(q, self0, dis, invd, gw1, gb0)

    r = _msg_kernel(g1, src2, dst2, w2)
    return _final(r, self1, dis, gb1)


# speedup vs baseline: 7.2582x; 1.1102x over previous
import functools

import jax
import jax.numpy as jnp
from jax import lax
from jax.experimental import pallas as pl
from jax.experimental.pallas import tpu as pltpu
from jax.experimental.pallas import tpu_sc as plsc

N = 10000
E = 320000
H = 128
NPAD = 10240
CH = 80
NW = 32
EPAD = 327680
EPW = EPAD // NW
NCH = EPW // CH
NB = 4
RPS = NPAD // 16


def _elu(v):
    return jnp.where(v > 0, v, jnp.exp(jnp.minimum(v, 0.0)) - 1.0)


def _edge_mlp_body(a_ref, ew0_ref, eb0_ref, ew1r_ref, eb1_ref, o_ref):
    a = a_ref[...]
    h1 = _elu(a * ew0_ref[...] + eb0_ref[...])
    w = jnp.sum(h1 * ew1r_ref[...], axis=1, keepdims=True) + eb1_ref[...]
    o_ref[...] = _elu(w)


def _edge_mlp(edge_attr, ew0, eb0, ew1, eb1):
    BE = 2000
    grid = (E // BE,)
    return pl.pallas_call(
        _edge_mlp_body,
        grid=grid,
        in_specs=[
            pl.BlockSpec((BE, 1), lambda i: (i, 0)),
            pl.BlockSpec((1, H), lambda i: (0, 0)),
            pl.BlockSpec((1, H), lambda i: (0, 0)),
            pl.BlockSpec((1, H), lambda i: (0, 0)),
            pl.BlockSpec((1, 1), lambda i: (0, 0)),
        ],
        out_specs=pl.BlockSpec((BE, 1), lambda i: (i, 0)),
        out_shape=jax.ShapeDtypeStruct((E, 1), jnp.float32),
    )(edge_attr, ew0, eb0.reshape(1, H), ew1.reshape(1, H), eb1.reshape(1, 1))


def _prep_body(x_ref, nw0_ref, nb0_ref, nw1_ref, nb1_ref, gw0_ref,
               dp0_ref, dp1_ref, g_ref, self_ref, dis_ref, invd_ref):
    h = _elu(jnp.dot(x_ref[...], nw0_ref[...],
                     preferred_element_type=jnp.float32) + nb0_ref[...])
    h = _elu(jnp.dot(h, nw1_ref[...],
                     preferred_element_type=jnp.float32) + nb1_ref[...])
    hw = jnp.dot(h, gw0_ref[...], preferred_element_type=jnp.float32)
    deg = 1.0 + dp0_ref[...] + dp1_ref[...]
    pos = deg > 0
    dis = jnp.where(pos, lax.rsqrt(jnp.abs(deg)), 0.0)
    invd = jnp.where(pos, 1.0 / deg, 0.0)
    g_ref[...] = hw * dis
    self_ref[...] = hw * invd
    dis_ref[...] = dis
    invd_ref[...] = invd


def _prep(x, nw0, nb0, nw1, nb1, gw0, dp0, dp1):
    BR = 400
    grid = (N // BR,)
    r = pl.BlockSpec((BR, H), lambda i: (i, 0))
    wsp = pl.BlockSpec((H, H), lambda i: (0, 0))
    b = pl.BlockSpec((1, H), lambda i: (0, 0))
    c = pl.BlockSpec((BR, 1), lambda i: (i, 0))
    return pl.pallas_call(
        _prep_body,
        grid=grid,
        in_specs=[r, wsp, b, wsp, b, wsp, c, c],
        out_specs=[r, r, c, c],
        out_shape=[
            jax.ShapeDtypeStruct((N, H), jnp.float32),
            jax.ShapeDtypeStruct((N, H), jnp.float32),
            jax.ShapeDtypeStruct((N, 1), jnp.float32),
            jax.ShapeDtypeStruct((N, 1), jnp.float32),
        ],
    )(x, nw0, nb0.reshape(1, H), nw1, nb1.reshape(1, H), gw0, dp0, dp1)


def _mid_body(q0_ref, q1_ref, self_ref, dis_ref, invd_ref, gw_ref, gb_ref,
              g_ref, selfo_ref):
    msum = q0_ref[0] + q1_ref[0]
    out1 = _elu(dis_ref[...] * msum + self_ref[...] + gb_ref[...])
    hw = jnp.dot(out1, gw_ref[...], preferred_element_type=jnp.float32)
    g_ref[...] = hw * dis_ref[...]
    selfo_ref[...] = hw * invd_ref[...]


def _mid(q, self0, dis, invd, gw, gb):
    BR = 400
    grid = (N // BR,)
    q0 = pl.BlockSpec((1, BR, H), lambda i: (0, i, 0))
    q1 = pl.BlockSpec((1, BR, H), lambda i: (1, i, 0))
    r = pl.BlockSpec((BR, H), lambda i: (i, 0))
    wsp = pl.BlockSpec((H, H), lambda i: (0, 0))
    b = pl.BlockSpec((1, H), lambda i: (0, 0))
    c = pl.BlockSpec((BR, 1), lambda i: (i, 0))
    return pl.pallas_call(
        _mid_body,
        grid=grid,
        in_specs=[q0, q1, r, c, c, wsp, b],
        out_specs=[r, r],
        out_shape=[
            jax.ShapeDtypeStruct((N, H), jnp.float32),
            jax.ShapeDtypeStruct((N, H), jnp.float32),
        ],
    )(q, q, self0, dis, invd, gw, gb.reshape(1, H))


def _final_body(q0_ref, q1_ref, self_ref, dis_ref, gb_ref, o_ref):
    msum = q0_ref[0] + q1_ref[0]
    o_ref[...] = _elu(dis_ref[...] * msum + self_ref[...] + gb_ref[...])


def _final(q, self1, dis, gb):
    BR = 400
    grid = (N // BR,)
    q0 = pl.BlockSpec((1, BR, H), lambda i: (0, i, 0))
    q1 = pl.BlockSpec((1, BR, H), lambda i: (1, i, 0))
    r = pl.BlockSpec((BR, H), lambda i: (i, 0))
    b = pl.BlockSpec((1, H), lambda i: (0, 0))
    c = pl.BlockSpec((BR, 1), lambda i: (i, 0))
    return pl.pallas_call(
        _final_body,
        grid=grid,
        in_specs=[q0, q1, r, c, b],
        out_specs=r,
        out_shape=jax.ShapeDtypeStruct((N, H), jnp.float32),
    )(q, q, self1, dis, gb.reshape(1, H))


_MESH = plsc.VectorSubcoreMesh(core_axis_name="c", subcore_axis_name="s")


@functools.partial(
    pl.kernel,
    mesh=_MESH,
    out_type=jax.ShapeDtypeStruct((2, NPAD), jnp.float32),
    scratch_types=[
        pltpu.VMEM((NCH, CH), jnp.int32),
        pltpu.VMEM((NCH, CH), jnp.float32),
        pltpu.VMEM((CH,), jnp.float32),
        pltpu.VMEM_SHARED((NPAD,), jnp.float32),
    ],
)
def _deg_kernel(dst_hbm, w_hbm, out_hbm, didx_v, w_v, zbuf, acc_sh):
    cid = lax.axis_index("c")
    sid = lax.axis_index("s")
    ws = cid * 16 + sid
    pltpu.sync_copy(dst_hbm.at[pl.ds(ws * NCH, NCH)], didx_v)
    pltpu.sync_copy(w_hbm.at[pl.ds(ws * NCH, NCH)], w_v)
    for j in range(CH // 16):
        zbuf[pl.ds(j * 16, 16)] = jnp.zeros((16,), jnp.float32)
    for k in range(RPS // CH):
        pltpu.sync_copy(zbuf, acc_sh.at[pl.ds(sid * RPS + k * CH, CH)])
    plsc.subcore_barrier()

    def body(c, carry):
        pltpu.sync_copy(w_v.at[c], acc_sh.at[didx_v.at[c]], add=True)
        return carry

    lax.fori_loop(0, NCH, body, 0)
    plsc.subcore_barrier()
    pltpu.sync_copy(acc_sh.at[pl.ds(sid * RPS, RPS)],
                    out_hbm.at[cid, pl.ds(sid * RPS, RPS)])


@functools.partial(
    pl.kernel,
    mesh=_MESH,
    out_type=jax.ShapeDtypeStruct((2, NPAD, H), jnp.float32),
    scratch_types=[
        pltpu.VMEM((8, CH), jnp.int32),
        pltpu.VMEM((8, CH), jnp.int32),
        pltpu.VMEM((8, CH), jnp.float32),
        pltpu.VMEM((NB, CH, H), jnp.float32),
        pltpu.VMEM_SHARED((NPAD, H), jnp.float32),
        [pltpu.SemaphoreType.DMA] * NB,
        [pltpu.SemaphoreType.DMA] * NB,
        [pltpu.SemaphoreType.DMA] * 2,
    ],
)
def _msg_kernel(g_hbm, src_hbm, dst_hbm, w_hbm, out_hbm,
                sidx_v, didx_v, w_v, rows_v, acc_sh, gsem, ssem, isem):
    cid = lax.axis_index("c")
    sid = lax.axis_index("s")
    ws = cid * 16 + sid
    ebase = ws * EPW

    def idx_load(c, kb, si):
        off = ebase + c * CH
        pltpu.async_copy(src_hbm.at[pl.ds(off, CH)], sidx_v.at[kb], isem[si])
        pltpu.async_copy(dst_hbm.at[pl.ds(off, CH)], didx_v.at[kb], isem[si])
        pltpu.async_copy(w_hbm.at[pl.ds(off, CH)], w_v.at[kb], isem[si])

    def idx_wait(c, kb, si):
        off = ebase + c * CH
        pltpu.make_async_copy(src_hbm.at[pl.ds(off, CH)], sidx_v.at[kb],
                              isem[si]).wait()
        pltpu.make_async_copy(dst_hbm.at[pl.ds(off, CH)], didx_v.at[kb],
                              isem[si]).wait()
        pltpu.make_async_copy(w_hbm.at[pl.ds(off, CH)], w_v.at[kb],
                              isem[si]).wait()

    def gather_start(kb, b):
        pltpu.async_copy(g_hbm.at[sidx_v.at[kb]], rows_v.at[b], gsem[b])

    def gather_wait(kb, b):
        pltpu.make_async_copy(g_hbm.at[sidx_v.at[kb]], rows_v.at[b],
                              gsem[b]).wait()

    def scatter_start(kb, b):
        pltpu.async_copy(rows_v.at[b], acc_sh.at[didx_v.at[kb]], ssem[b],
                         add=True)

    def scatter_wait(kb, b):
        pltpu.make_async_copy(rows_v.at[b], acc_sh.at[didx_v.at[kb]],
                              ssem[b]).wait()

    def scale(kb, b):
        def rbody(eg, rc):
            wvec = w_v[kb, pl.ds(eg * 16, 16)]
            for i in range(16):
                sc = wvec[i]
                e = eg * 16 + i
                for j in range(H // 16):
                    rows_v[b, e, pl.ds(j * 16, 16)] = (
                        rows_v[b, e, pl.ds(j * 16, 16)] * sc)
            return rc

        lax.fori_loop(0, CH // 16, rbody, 0)

    def zrow(r, carry):
        for j in range(H // 16):
            rows_v[0, r, pl.ds(j * 16, 16)] = jnp.zeros((16,), jnp.float32)
        return carry

    lax.fori_loop(0, CH, zrow, 0)
    for k in range(RPS // CH):
        pltpu.sync_copy(rows_v.at[0], acc_sh.at[pl.ds(sid * RPS + k * CH, CH)])
    plsc.subcore_barrier()

    idx_load(0, 0, 0)
    idx_wait(0, 0, 0)
    gather_start(0, 0)
    idx_load(1, 1, 1)

    def grp(g8, carry):
        for k in range(8):
            c = g8 * 8 + k
            b = k % NB

            @pl.when(c >= 3)
            def _():
                scatter_wait((k - 3) % 8, (k - 3) % NB)

            @pl.when(c + 1 < NCH)
            def _():
                idx_wait(c + 1, (k + 1) % 8, (k + 1) % 2)
                gather_start((k + 1) % 8, (k + 1) % NB)

            @pl.when(c + 2 < NCH)
            def _():
                idx_load(c + 2, (k + 2) % 8, k % 2)

            gather_wait(k, b)
            scale(k, b)
            scatter_start(k, b)
        return carry

    lax.fori_loop(0, NCH // 8, grp, 0)
    for c in (NCH - 3, NCH - 2, NCH - 1):
        scatter_wait(c % 8, c % NB)
    plsc.subcore_barrier()
    pltpu.sync_copy(acc_sh.at[pl.ds(sid * RPS, RPS)],
                    out_hbm.at[cid, pl.ds(sid * RPS, RPS)])


def kernel(x, edge_index, edge_attr, pos, nw0, nb0, nw1, nb1,
           ew0, eb0, ew1, eb1, gw0, gb0, gw1, gb1):
    pad = jnp.zeros((EPAD - E,), jnp.int32)
    src1 = jnp.concatenate([edge_index[0], pad])
    dst1 = jnp.concatenate([edge_index[1], pad])
    dst2 = dst1.reshape(EPAD // CH, CH)

    w_edge = _edge_mlp(edge_attr, ew0, eb0, ew1, eb1)
    w1 = jnp.concatenate(
        [w_edge.reshape(E), jnp.zeros((EPAD - E,), jnp.float32)])
    w2 = w1.reshape(EPAD // CH, CH)

    degp = _deg_kernel(dst2, w2)
    dp0 = degp[0, :N].reshape(N, 1)
    dp1 = degp[1, :N].reshape(N, 1)

    g0, self0, dis, invd = _prep(x, nw0, nb0, nw1, nb1, gw0, dp0, dp1)

    q = _msg_kernel(g0, src1, dst1, w1)
    g1, self1 = _mid(q, self0, dis, invd, gw1, gb0)

    r = _msg_kernel(g1, src1, dst1, w1)
    return _final(r, self1, dis, gb1)


# speedup vs baseline: 7.3025x; 1.0061x over previous
import functools

import jax
import jax.numpy as jnp
from jax import lax
from jax.experimental import pallas as pl
from jax.experimental.pallas import tpu as pltpu
from jax.experimental.pallas import tpu_sc as plsc

N = 10000
E = 320000
H = 128
NPAD = 10240
CH = 80
NW = 32
EPAD = 327680
EPW = EPAD // NW
NCH = EPW // CH
NB = 4
RPS = NPAD // 16


def _elu(v):
    return jnp.where(v > 0, v, jnp.exp(jnp.minimum(v, 0.0)) - 1.0)


def _edge_mlp_body(a_ref, ew0_ref, eb0_ref, ew1r_ref, eb1_ref, o_ref):
    a = a_ref[...]
    h1 = _elu(a * ew0_ref[...] + eb0_ref[...])
    w = jnp.sum(h1 * ew1r_ref[...], axis=1, keepdims=True) + eb1_ref[...]
    o_ref[...] = _elu(w)


def _edge_mlp(edge_attr, ew0, eb0, ew1, eb1):
    BE = 2000
    grid = (E // BE,)
    return pl.pallas_call(
        _edge_mlp_body,
        grid=grid,
        in_specs=[
            pl.BlockSpec((BE, 1), lambda i: (i, 0)),
            pl.BlockSpec((1, H), lambda i: (0, 0)),
            pl.BlockSpec((1, H), lambda i: (0, 0)),
            pl.BlockSpec((1, H), lambda i: (0, 0)),
            pl.BlockSpec((1, 1), lambda i: (0, 0)),
        ],
        out_specs=pl.BlockSpec((BE, 1), lambda i: (i, 0)),
        out_shape=jax.ShapeDtypeStruct((E, 1), jnp.float32),
    )(edge_attr, ew0, eb0.reshape(1, H), ew1.reshape(1, H), eb1.reshape(1, 1))


def _prep_body(x_ref, nw0_ref, nb0_ref, nw1_ref, nb1_ref, gw0_ref,
               dp0_ref, dp1_ref, g_ref, self_ref, dis_ref, invd_ref):
    h = _elu(jnp.dot(x_ref[...], nw0_ref[...],
                     preferred_element_type=jnp.float32) + nb0_ref[...])
    h = _elu(jnp.dot(h, nw1_ref[...],
                     preferred_element_type=jnp.float32) + nb1_ref[...])
    hw = jnp.dot(h, gw0_ref[...], preferred_element_type=jnp.float32)
    deg = 1.0 + dp0_ref[...] + dp1_ref[...]
    pos = deg > 0
    dis = jnp.where(pos, lax.rsqrt(jnp.abs(deg)), 0.0)
    invd = jnp.where(pos, 1.0 / deg, 0.0)
    g_ref[...] = hw * dis
    self_ref[...] = hw * invd
    dis_ref[...] = dis
    invd_ref[...] = invd


def _prep(x, nw0, nb0, nw1, nb1, gw0, dp0, dp1):
    BR = 400
    grid = (N // BR,)
    r = pl.BlockSpec((BR, H), lambda i: (i, 0))
    wsp = pl.BlockSpec((H, H), lambda i: (0, 0))
    b = pl.BlockSpec((1, H), lambda i: (0, 0))
    c = pl.BlockSpec((BR, 1), lambda i: (i, 0))
    return pl.pallas_call(
        _prep_body,
        grid=grid,
        in_specs=[r, wsp, b, wsp, b, wsp, c, c],
        out_specs=[r, r, c, c],
        out_shape=[
            jax.ShapeDtypeStruct((N, H), jnp.float32),
            jax.ShapeDtypeStruct((N, H), jnp.float32),
            jax.ShapeDtypeStruct((N, 1), jnp.float32),
            jax.ShapeDtypeStruct((N, 1), jnp.float32),
        ],
    )(x, nw0, nb0.reshape(1, H), nw1, nb1.reshape(1, H), gw0, dp0, dp1)


def _mid_body(q0_ref, q1_ref, self_ref, dis_ref, invd_ref, gw_ref, gb_ref,
              g_ref, selfo_ref):
    msum = q0_ref[0] + q1_ref[0]
    out1 = _elu(dis_ref[...] * msum + self_ref[...] + gb_ref[...])
    hw = jnp.dot(out1, gw_ref[...], preferred_element_type=jnp.float32)
    g_ref[...] = hw * dis_ref[...]
    selfo_ref[...] = hw * invd_ref[...]


def _mid(q, self0, dis, invd, gw, gb):
    BR = 400
    grid = (N // BR,)
    q0 = pl.BlockSpec((1, BR, H), lambda i: (0, i, 0))
    q1 = pl.BlockSpec((1, BR, H), lambda i: (1, i, 0))
    r = pl.BlockSpec((BR, H), lambda i: (i, 0))
    wsp = pl.BlockSpec((H, H), lambda i: (0, 0))
    b = pl.BlockSpec((1, H), lambda i: (0, 0))
    c = pl.BlockSpec((BR, 1), lambda i: (i, 0))
    return pl.pallas_call(
        _mid_body,
        grid=grid,
        in_specs=[q0, q1, r, c, c, wsp, b],
        out_specs=[r, r],
        out_shape=[
            jax.ShapeDtypeStruct((N, H), jnp.float32),
            jax.ShapeDtypeStruct((N, H), jnp.float32),
        ],
    )(q, q, self0, dis, invd, gw, gb.reshape(1, H))


def _final_body(q0_ref, q1_ref, self_ref, dis_ref, gb_ref, o_ref):
    msum = q0_ref[0] + q1_ref[0]
    o_ref[...] = _elu(dis_ref[...] * msum + self_ref[...] + gb_ref[...])


def _final(q, self1, dis, gb):
    BR = 400
    grid = (N // BR,)
    q0 = pl.BlockSpec((1, BR, H), lambda i: (0, i, 0))
    q1 = pl.BlockSpec((1, BR, H), lambda i: (1, i, 0))
    r = pl.BlockSpec((BR, H), lambda i: (i, 0))
    b = pl.BlockSpec((1, H), lambda i: (0, 0))
    c = pl.BlockSpec((BR, 1), lambda i: (i, 0))
    return pl.pallas_call(
        _final_body,
        grid=grid,
        in_specs=[q0, q1, r, c, b],
        out_specs=r,
        out_shape=jax.ShapeDtypeStruct((N, H), jnp.float32),
    )(q, q, self1, dis, gb.reshape(1, H))


_MESH = plsc.VectorSubcoreMesh(core_axis_name="c", subcore_axis_name="s")


@functools.partial(
    pl.kernel,
    mesh=_MESH,
    out_type=jax.ShapeDtypeStruct((2, NPAD), jnp.float32),
    scratch_types=[
        pltpu.VMEM((NCH, CH), jnp.int32),
        pltpu.VMEM((NCH, CH), jnp.float32),
        pltpu.VMEM((CH,), jnp.float32),
        pltpu.VMEM_SHARED((NPAD,), jnp.float32),
    ],
)
def _deg_kernel(dst_hbm, w_hbm, out_hbm, didx_v, w_v, zbuf, acc_sh):
    cid = lax.axis_index("c")
    sid = lax.axis_index("s")
    ws = cid * 16 + sid
    pltpu.sync_copy(dst_hbm.at[pl.ds(ws * NCH, NCH)], didx_v)
    pltpu.sync_copy(w_hbm.at[pl.ds(ws * NCH, NCH)], w_v)
    for j in range(CH // 16):
        zbuf[pl.ds(j * 16, 16)] = jnp.zeros((16,), jnp.float32)
    for k in range(RPS // CH):
        pltpu.sync_copy(zbuf, acc_sh.at[pl.ds(sid * RPS + k * CH, CH)])
    plsc.subcore_barrier()

    def body(c, carry):
        pltpu.sync_copy(w_v.at[c], acc_sh.at[didx_v.at[c]], add=True)
        return carry

    lax.fori_loop(0, NCH, body, 0)
    plsc.subcore_barrier()
    pltpu.sync_copy(acc_sh.at[pl.ds(sid * RPS, RPS)],
                    out_hbm.at[cid, pl.ds(sid * RPS, RPS)])


@functools.partial(
    pl.kernel,
    mesh=_MESH,
    out_type=jax.ShapeDtypeStruct((2, NPAD, H), jnp.float32),
    scratch_types=[
        pltpu.VMEM((8, CH), jnp.int32),
        pltpu.VMEM((8, CH), jnp.int32),
        pltpu.VMEM((8, CH), jnp.float32),
        pltpu.VMEM((NB, CH, H), jnp.float32),
        pltpu.VMEM_SHARED((NPAD, H), jnp.float32),
        [pltpu.SemaphoreType.DMA] * NB,
        [pltpu.SemaphoreType.DMA] * NB,
        [pltpu.SemaphoreType.DMA] * 2,
    ],
)
def _msg_kernel(g_hbm, src_hbm, dst_hbm, w_hbm, out_hbm,
                sidx_v, didx_v, w_v, rows_v, acc_sh, gsem, ssem, isem):
    cid = lax.axis_index("c")
    sid = lax.axis_index("s")
    ws = cid * 16 + sid
    ebase = ws * EPW

    def idx_load(c, kb, si):
        off = ebase + c * CH
        pltpu.async_copy(src_hbm.at[pl.ds(off, CH)], sidx_v.at[kb], isem[si])
        pltpu.async_copy(dst_hbm.at[pl.ds(off, CH)], didx_v.at[kb], isem[si])
        pltpu.async_copy(w_hbm.at[pl.ds(off, CH)], w_v.at[kb], isem[si])

    def idx_wait(c, kb, si):
        off = ebase + c * CH
        pltpu.make_async_copy(src_hbm.at[pl.ds(off, CH)], sidx_v.at[kb],
                              isem[si]).wait()
        pltpu.make_async_copy(dst_hbm.at[pl.ds(off, CH)], didx_v.at[kb],
                              isem[si]).wait()
        pltpu.make_async_copy(w_hbm.at[pl.ds(off, CH)], w_v.at[kb],
                              isem[si]).wait()

    def gather_start(kb, b):
        pltpu.async_copy(g_hbm.at[sidx_v.at[kb]], rows_v.at[b], gsem[b])

    def gather_wait(kb, b):
        pltpu.make_async_copy(g_hbm.at[sidx_v.at[kb]], rows_v.at[b],
                              gsem[b]).wait()

    def scatter_start(kb, b):
        pltpu.async_copy(rows_v.at[b], acc_sh.at[didx_v.at[kb]], ssem[b],
                         add=True)

    def scatter_wait(kb, b):
        pltpu.make_async_copy(rows_v.at[b], acc_sh.at[didx_v.at[kb]],
                              ssem[b]).wait()

    def scale(kb, b):
        def rbody(eg, rc):
            wvec = w_v[kb, pl.ds(eg * 16, 16)]
            for i in range(16):
                sc = wvec[i]
                e = eg * 16 + i
                for j in range(H // 16):
                    rows_v[b, e, pl.ds(j * 16, 16)] = (
                        rows_v[b, e, pl.ds(j * 16, 16)] * sc)
            return rc

        lax.fori_loop(0, CH // 16, rbody, 0)

    def zrow(r, carry):
        for j in range(H // 16):
            rows_v[0, r, pl.ds(j * 16, 16)] = jnp.zeros((16,), jnp.float32)
        return carry

    lax.fori_loop(0, CH, zrow, 0)
    for k in range(RPS // CH):
        pltpu.sync_copy(rows_v.at[0], acc_sh.at[pl.ds(sid * RPS + k * CH, CH)])
    plsc.subcore_barrier()

    idx_load(0, 0, 0)
    idx_wait(0, 0, 0)
    gather_start(0, 0)
    idx_load(1, 1, 1)

    def grp(g8, carry):
        for k in range(8):
            c = g8 * 8 + k
            b = k % NB

            @pl.when(c >= 3)
            def _():
                scatter_wait((k - 3) % 8, (k - 3) % NB)

            @pl.when(c + 1 < NCH)
            def _():
                idx_wait(c + 1, (k + 1) % 8, (k + 1) % 2)
                gather_start((k + 1) % 8, (k + 1) % NB)

            @pl.when(c + 2 < NCH)
            def _():
                idx_load(c + 2, (k + 2) % 8, k % 2)

            gather_wait(k, b)
            scale(k, b)
            scatter_start(k, b)
        return carry

    lax.fori_loop(0, NCH // 8, grp, 0)
    for c in (5, 6, 7):
        scatter_wait(c, c % NB)
    plsc.subcore_barrier()
    pltpu.sync_copy(acc_sh.at[pl.ds(sid * RPS, RPS)],
                    out_hbm.at[cid, pl.ds(sid * RPS, RPS)])


def kernel(x, edge_index, edge_attr, pos, nw0, nb0, nw1, nb1,
           ew0, eb0, ew1, eb1, gw0, gb0, gw1, gb1):
    pad = jnp.zeros((EPAD - E,), jnp.int32)
    src1 = jnp.concatenate([edge_index[0], pad])
    dst1 = jnp.concatenate([edge_index[1], pad])
    dst2 = dst1.reshape(EPAD // CH, CH)

    w_edge = _edge_mlp(edge_attr, ew0, eb0, ew1, eb1)
    w1 = jnp.concatenate(
        [w_edge.reshape(E), jnp.zeros((EPAD - E,), jnp.float32)])
    w2 = w1.reshape(EPAD // CH, CH)

    degp = _deg_kernel(dst2, w2)
    dp0 = degp[0, :N].reshape(N, 1)
    dp1 = degp[1, :N].reshape(N, 1)

    g0, self0, dis, invd = _prep(x, nw0, nb0, nw1, nb1, gw0, dp0, dp1)

    q = _msg_kernel(g0, src1, dst1, w1)
    g1, self1 = _mid(q, self0, dis, invd, gw1, gb0)

    r = _msg_kernel(g1, src1, dst1, w1)
    return _final(r, self1, dis, gb1)


# speedup vs baseline: 8.3570x; 1.1444x over previous
import functools

import jax
import jax.numpy as jnp
from jax import lax
from jax.experimental import pallas as pl
from jax.experimental.pallas import tpu as pltpu
from jax.experimental.pallas import tpu_sc as plsc

N = 10000
E = 320000
H = 128
NPAD = 10240
CH = 80
NW = 32
EPAD = 327680
EPW = EPAD // NW
NCH = EPW // CH
NB = 4
RPS = NPAD // 16


def _elu(v):
    return jnp.where(v > 0, v, jnp.exp(jnp.minimum(v, 0.0)) - 1.0)


def _prep_body(x_ref, nw0_ref, nb0_ref, nw1_ref, nb1_ref, gw0_ref,
               dp0_ref, dp1_ref, g_ref, self_ref, dis_ref, invd_ref):
    h = _elu(jnp.dot(x_ref[...], nw0_ref[...],
                     preferred_element_type=jnp.float32) + nb0_ref[...])
    h = _elu(jnp.dot(h, nw1_ref[...],
                     preferred_element_type=jnp.float32) + nb1_ref[...])
    hw = jnp.dot(h, gw0_ref[...], preferred_element_type=jnp.float32)
    deg = 1.0 + dp0_ref[...] + dp1_ref[...]
    pos = deg > 0
    dis = jnp.where(pos, lax.rsqrt(jnp.abs(deg)), 0.0)
    invd = jnp.where(pos, 1.0 / deg, 0.0)
    g_ref[...] = hw * dis
    self_ref[...] = hw * invd
    dis_ref[...] = dis
    invd_ref[...] = invd


def _prep(x, nw0, nb0, nw1, nb1, gw0, dp0, dp1):
    BR = 400
    grid = (N // BR,)
    r = pl.BlockSpec((BR, H), lambda i: (i, 0))
    wsp = pl.BlockSpec((H, H), lambda i: (0, 0))
    b = pl.BlockSpec((1, H), lambda i: (0, 0))
    c = pl.BlockSpec((BR, 1), lambda i: (i, 0))
    return pl.pallas_call(
        _prep_body,
        grid=grid,
        in_specs=[r, wsp, b, wsp, b, wsp, c, c],
        out_specs=[r, r, c, c],
        out_shape=[
            jax.ShapeDtypeStruct((N, H), jnp.float32),
            jax.ShapeDtypeStruct((N, H), jnp.float32),
            jax.ShapeDtypeStruct((N, 1), jnp.float32),
            jax.ShapeDtypeStruct((N, 1), jnp.float32),
        ],
    )(x, nw0, nb0.reshape(1, H), nw1, nb1.reshape(1, H), gw0, dp0, dp1)


def _mid_body(q0_ref, q1_ref, self_ref, dis_ref, invd_ref, gw_ref, gb_ref,
              g_ref, selfo_ref):
    msum = q0_ref[0] + q1_ref[0]
    out1 = _elu(dis_ref[...] * msum + self_ref[...] + gb_ref[...])
    hw = jnp.dot(out1, gw_ref[...], preferred_element_type=jnp.float32)
    g_ref[...] = hw * dis_ref[...]
    selfo_ref[...] = hw * invd_ref[...]


def _mid(q, self0, dis, invd, gw, gb):
    BR = 400
    grid = (N // BR,)
    q0 = pl.BlockSpec((1, BR, H), lambda i: (0, i, 0))
    q1 = pl.BlockSpec((1, BR, H), lambda i: (1, i, 0))
    r = pl.BlockSpec((BR, H), lambda i: (i, 0))
    wsp = pl.BlockSpec((H, H), lambda i: (0, 0))
    b = pl.BlockSpec((1, H), lambda i: (0, 0))
    c = pl.BlockSpec((BR, 1), lambda i: (i, 0))
    return pl.pallas_call(
        _mid_body,
        grid=grid,
        in_specs=[q0, q1, r, c, c, wsp, b],
        out_specs=[r, r],
        out_shape=[
            jax.ShapeDtypeStruct((N, H), jnp.float32),
            jax.ShapeDtypeStruct((N, H), jnp.float32),
        ],
    )(q, q, self0, dis, invd, gw, gb.reshape(1, H))


def _final_body(q0_ref, q1_ref, self_ref, dis_ref, gb_ref, o_ref):
    msum = q0_ref[0] + q1_ref[0]
    o_ref[...] = _elu(dis_ref[...] * msum + self_ref[...] + gb_ref[...])


def _final(q, self1, dis, gb):
    BR = 400
    grid = (N // BR,)
    q0 = pl.BlockSpec((1, BR, H), lambda i: (0, i, 0))
    q1 = pl.BlockSpec((1, BR, H), lambda i: (1, i, 0))
    r = pl.BlockSpec((BR, H), lambda i: (i, 0))
    b = pl.BlockSpec((1, H), lambda i: (0, 0))
    c = pl.BlockSpec((BR, 1), lambda i: (i, 0))
    return pl.pallas_call(
        _final_body,
        grid=grid,
        in_specs=[q0, q1, r, c, b],
        out_specs=r,
        out_shape=jax.ShapeDtypeStruct((N, H), jnp.float32),
    )(q, q, self1, dis, gb.reshape(1, H))


_MESH = plsc.VectorSubcoreMesh(core_axis_name="c", subcore_axis_name="s")


@functools.partial(
    pl.kernel,
    mesh=_MESH,
    out_type=jax.ShapeDtypeStruct((2, NPAD), jnp.float32),
    scratch_types=[
        pltpu.VMEM((NCH, CH), jnp.int32),
        pltpu.VMEM((NCH, CH), jnp.float32),
        pltpu.VMEM((CH,), jnp.float32),
        pltpu.VMEM_SHARED((NPAD,), jnp.float32),
    ],
)
def _deg_kernel(dst_hbm, w_hbm, out_hbm, didx_v, w_v, zbuf, acc_sh):
    cid = lax.axis_index("c")
    sid = lax.axis_index("s")
    ws = cid * 16 + sid
    pltpu.sync_copy(dst_hbm.at[pl.ds(ws * NCH, NCH)], didx_v)
    pltpu.sync_copy(w_hbm.at[pl.ds(ws * NCH, NCH)], w_v)
    for j in range(CH // 16):
        zbuf[pl.ds(j * 16, 16)] = jnp.zeros((16,), jnp.float32)
    for k in range(RPS // CH):
        pltpu.sync_copy(zbuf, acc_sh.at[pl.ds(sid * RPS + k * CH, CH)])
    plsc.subcore_barrier()

    def body(c, carry):
        pltpu.sync_copy(w_v.at[c], acc_sh.at[didx_v.at[c]], add=True)
        return carry

    lax.fori_loop(0, NCH, body, 0)
    plsc.subcore_barrier()
    pltpu.sync_copy(acc_sh.at[pl.ds(sid * RPS, RPS)],
                    out_hbm.at[cid, pl.ds(sid * RPS, RPS)])


@functools.partial(
    pl.kernel,
    mesh=_MESH,
    out_type=jax.ShapeDtypeStruct((2, NPAD, H), jnp.float32),
    scratch_types=[
        pltpu.VMEM((8, CH), jnp.int32),
        pltpu.VMEM((8, CH), jnp.int32),
        pltpu.VMEM((8, CH), jnp.float32),
        pltpu.VMEM((NB, CH, H), jnp.float32),
        pltpu.VMEM_SHARED((NPAD, H), jnp.float32),
        [pltpu.SemaphoreType.DMA] * NB,
        [pltpu.SemaphoreType.DMA] * NB,
        [pltpu.SemaphoreType.DMA] * 2,
    ],
)
def _msg_kernel(g_hbm, src_hbm, dst_hbm, w_hbm, out_hbm,
                sidx_v, didx_v, w_v, rows_v, acc_sh, gsem, ssem, isem):
    cid = lax.axis_index("c")
    sid = lax.axis_index("s")
    ws = cid * 16 + sid
    ebase = ws * EPW

    def idx_load(c, kb, si):
        off = ebase + c * CH
        pltpu.async_copy(src_hbm.at[pl.ds(off, CH)], sidx_v.at[kb], isem[si])
        pltpu.async_copy(dst_hbm.at[pl.ds(off, CH)], didx_v.at[kb], isem[si])
        pltpu.async_copy(w_hbm.at[pl.ds(off, CH)], w_v.at[kb], isem[si])

    def idx_wait(c, kb, si):
        off = ebase + c * CH
        pltpu.make_async_copy(src_hbm.at[pl.ds(off, CH)], sidx_v.at[kb],
                              isem[si]).wait()
        pltpu.make_async_copy(dst_hbm.at[pl.ds(off, CH)], didx_v.at[kb],
                              isem[si]).wait()
        pltpu.make_async_copy(w_hbm.at[pl.ds(off, CH)], w_v.at[kb],
                              isem[si]).wait()

    def gather_start(kb, b):
        pltpu.async_copy(g_hbm.at[sidx_v.at[kb]], rows_v.at[b], gsem[b])

    def gather_wait(kb, b):
        pltpu.make_async_copy(g_hbm.at[sidx_v.at[kb]], rows_v.at[b],
                              gsem[b]).wait()

    def scatter_start(kb, b):
        pltpu.async_copy(rows_v.at[b], acc_sh.at[didx_v.at[kb]], ssem[b],
                         add=True)

    def scatter_wait(kb, b):
        pltpu.make_async_copy(rows_v.at[b], acc_sh.at[didx_v.at[kb]],
                              ssem[b]).wait()

    def scale(kb, b):
        def rbody(eg, rc):
            wvec = w_v[kb, pl.ds(eg * 16, 16)]
            for i in range(16):
                sc = wvec[i]
                e = eg * 16 + i
                for j in range(H // 16):
                    rows_v[b, e, pl.ds(j * 16, 16)] = (
                        rows_v[b, e, pl.ds(j * 16, 16)] * sc)
            return rc

        lax.fori_loop(0, CH // 16, rbody, 0)

    def zrow(r, carry):
        for j in range(H // 16):
            rows_v[0, r, pl.ds(j * 16, 16)] = jnp.zeros((16,), jnp.float32)
        return carry

    lax.fori_loop(0, CH, zrow, 0)
    for k in range(RPS // CH):
        pltpu.sync_copy(rows_v.at[0], acc_sh.at[pl.ds(sid * RPS + k * CH, CH)])
    plsc.subcore_barrier()

    idx_load(0, 0, 0)
    idx_wait(0, 0, 0)
    gather_start(0, 0)
    idx_load(1, 1, 1)

    def grp(g8, carry):
        for k in range(8):
            c = g8 * 8 + k
            b = k % NB

            @pl.when(c >= 3)
            def _():
                scatter_wait((k - 3) % 8, (k - 3) % NB)

            @pl.when(c + 1 < NCH)
            def _():
                idx_wait(c + 1, (k + 1) % 8, (k + 1) % 2)
                gather_start((k + 1) % 8, (k + 1) % NB)

            @pl.when(c + 2 < NCH)
            def _():
                idx_load(c + 2, (k + 2) % 8, k % 2)

            gather_wait(k, b)
            scale(k, b)
            scatter_start(k, b)
        return carry

    lax.fori_loop(0, NCH // 8, grp, 0)
    for c in (5, 6, 7):
        scatter_wait(c, c % NB)
    plsc.subcore_barrier()
    pltpu.sync_copy(acc_sh.at[pl.ds(sid * RPS, RPS)],
                    out_hbm.at[cid, pl.ds(sid * RPS, RPS)])


def kernel(x, edge_index, edge_attr, pos, nw0, nb0, nw1, nb1,
           ew0, eb0, ew1, eb1, gw0, gb0, gw1, gb1):
    pad = jnp.zeros((EPAD - E,), jnp.int32)
    src1 = jnp.concatenate([edge_index[0], pad])
    dst1 = jnp.concatenate([edge_index[1], pad])
    dst2 = dst1.reshape(EPAD // CH, CH)

    ea = jax.nn.elu(edge_attr @ ew0 + eb0)
    ea = jax.nn.elu(ea @ ew1 + eb1)
    w1 = jnp.concatenate([ea[:, 0], jnp.zeros((EPAD - E,), jnp.float32)])
    w2 = w1.reshape(EPAD // CH, CH)

    degp = _deg_kernel(dst2, w2)
    dp0 = degp[0, :N].reshape(N, 1)
    dp1 = degp[1, :N].reshape(N, 1)

    g0, self0, dis, invd = _prep(x, nw0, nb0, nw1, nb1, gw0, dp0, dp1)

    q = _msg_kernel(g0, src1, dst1, w1)
    g1, self1 = _mid(q, self0, dis, invd, gw1, gb0)

    r = _msg_kernel(g1, src1, dst1, w1)
    return _final(r, self1, dis, gb1)
